# baseline (speedup 1.0000x reference)
import functools

import jax
import jax.numpy as jnp
import numpy as np
from jax.experimental import pallas as pl

_ANCHORS = np.array([[1.146, 1.621, 3.88],
                     [1.52, 1.93, 5.08],
                     [1.73, 2.58, 10.1]], dtype=np.float32)
_C = 12
_G = 76
_K = _G * _G
_N = 3 * _K
_STRIDE = 8.0


def _decode_kernel(x0_ref, x1_ref, x2_ref, o_ref):
    c = pl.program_id(0)
    B = x0_ref.shape[2]
    stk = jnp.stack([x0_ref[0], x1_ref[0], x2_ref[0]])
    v = jnp.transpose(stk, (2, 0, 1, 3)).reshape(B, _N)

    def sig(x):
        return jnp.tanh(x * 0.5) * 0.5 + 0.5

    def niota():
        return jax.lax.broadcasted_iota(jnp.int32, (B, _N), 1)

    @pl.when(c == 0)
    def _():
        gx = (niota() % _G).astype(jnp.float32) * _STRIDE
        o_ref[0] = sig(v) * _STRIDE + gx

    @pl.when(c == 1)
    def _():
        gy = ((niota() // _G) % _G).astype(jnp.float32) * _STRIDE
        o_ref[0] = sig(v) * _STRIDE + gy

    @pl.when((c == 2) | (c >= 8))
    def _():
        o_ref[0] = sig(v)

    @pl.when((c >= 3) & (c <= 5))
    def _():
        n = jax.lax.broadcasted_iota(jnp.int32, (B, _N), 1)
        def anchor_row(a):
            return jnp.where(c == 3, float(_ANCHORS[a, 0]),
                             jnp.where(c == 4, float(_ANCHORS[a, 1]),
                                       float(_ANCHORS[a, 2])))
        avec = jnp.where(n < _K, anchor_row(0),
                         jnp.where(n < 2 * _K, anchor_row(1), anchor_row(2)))
        o_ref[0] = jnp.minimum(jnp.exp(v), 1000.0) * avec

    @pl.when((c == 6) | (c == 7))
    def _():
        o_ref[0] = v


@jax.jit
def kernel(x):
    B = x.shape[0]
    xt = jnp.transpose(x, (1, 2, 0, 3))
    out = pl.pallas_call(
        _decode_kernel,
        grid=(_C,),
        in_specs=[
            pl.BlockSpec((1, _G, B, _G), lambda c, a=a: (c + _C * a, 0, 0, 0))
            for a in range(3)
        ],
        out_specs=pl.BlockSpec((1, B, _N), lambda c: (c, 0, 0)),
        out_shape=jax.ShapeDtypeStruct((_C, B, _N), jnp.float32),
    )(xt, xt, xt)
    return jnp.transpose(out, (1, 2, 0))
